# Initial kernel scaffold; baseline (speedup 1.0000x reference)
import jax
import jax.numpy as jnp
from jax.experimental import pallas as pl


def kernel(regression_heads, classification_heads, anchors):
    raise NotImplementedError("write your pallas kernel here")


# speedup vs baseline: 9.0395x; 9.0395x over previous
import math

import jax
import jax.numpy as jnp
from jax import lax
from jax.experimental import pallas as pl

_NUM_CLASSES = 2
_IMG_H, _IMG_W = 512.0, 512.0
_SCORE_THRESH = 0.05
_NMS_THRESH = 0.5
_TOPK = 1000
_DETS = 100
_LOG_MAX = math.log(1000.0 / 16)


def _decode(rel_boxes, anchors):
    anchors = anchors.astype(rel_boxes.dtype)
    widths = anchors[:, 2] - anchors[:, 0]
    heights = anchors[:, 3] - anchors[:, 1]
    ctr_x = anchors[:, 0] + 0.5 * widths
    ctr_y = anchors[:, 1] + 0.5 * heights
    dx = rel_boxes[:, 0::4]
    dy = rel_boxes[:, 1::4]
    dw = jnp.minimum(rel_boxes[:, 2::4], _LOG_MAX)
    dh = jnp.minimum(rel_boxes[:, 3::4], _LOG_MAX)
    pred_ctr_x = dx * widths[:, None] + ctr_x[:, None]
    pred_ctr_y = dy * heights[:, None] + ctr_y[:, None]
    pred_w = jnp.exp(dw) * widths[:, None]
    pred_h = jnp.exp(dh) * heights[:, None]
    c_to_c_w = 0.5 * pred_w
    c_to_c_h = 0.5 * pred_h
    pred = jnp.stack((pred_ctr_x - c_to_c_w, pred_ctr_y - c_to_c_h,
                      pred_ctr_x + c_to_c_w, pred_ctr_y + c_to_c_h), axis=2)
    return pred.reshape(rel_boxes.shape[0], -1)


def _clip_boxes(boxes):
    bx = jnp.clip(boxes[..., 0::2], 0.0, _IMG_W)
    by = jnp.clip(boxes[..., 1::2], 0.0, _IMG_H)
    return jnp.stack((bx, by), axis=boxes.ndim).reshape(boxes.shape)


def _nms_kernel(x1_ref, y1_ref, x2_ref, y2_ref, sc_ref, iv_ref,
                ox1_ref, oy1_ref, ox2_ref, oy2_ref, osc_ref, olab_ref):
    x1 = x1_ref[0]
    y1 = y1_ref[0]
    x2 = x2_ref[0]
    y2 = y2_ref[0]
    sc = sc_ref[0]
    areas = (x2 - x1) * (y2 - y1)
    row = lax.broadcasted_iota(jnp.int32, (16, 128), 0)
    lane2 = lax.broadcasted_iota(jnp.int32, (16, 128), 1)
    flat = row * 128 + lane2
    lane1 = lax.broadcasted_iota(jnp.int32, (1, 128), 1)
    active0 = iv_ref[0]
    zf = jnp.zeros((1, 128), jnp.float32)
    zi = jnp.zeros((1, 128), jnp.int32)

    def body(j, carry):
        active, first, ox1, oy1, ox2, oy2, osc, olab = carry
        masked = jnp.where(active > 0, flat, 4096)
        idx = jnp.min(masked)
        has = idx < 4096
        sel = jnp.where(has, idx, first)
        first = jnp.where(j == 0, sel, first)
        oh = flat == sel
        bx1 = jnp.max(jnp.where(oh, x1, -jnp.inf))
        by1 = jnp.max(jnp.where(oh, y1, -jnp.inf))
        bx2 = jnp.max(jnp.where(oh, x2, -jnp.inf))
        by2 = jnp.max(jnp.where(oh, y2, -jnp.inf))
        bsc = jnp.max(jnp.where(oh, sc, -jnp.inf))
        lab = jnp.where(sel >= _TOPK, 1, 0).astype(jnp.int32)
        wr = lane1 == j
        ox1 = jnp.where(wr, bx1, ox1)
        oy1 = jnp.where(wr, by1, oy1)
        ox2 = jnp.where(wr, bx2, ox2)
        oy2 = jnp.where(wr, by2, oy2)
        osc = jnp.where(wr, bsc, osc)
        olab = jnp.where(wr, lab, olab)
        idx0 = jnp.where(has, idx, 0)
        oh0 = flat == idx0
        sx1 = jnp.max(jnp.where(oh0, x1, -jnp.inf))
        sy1 = jnp.max(jnp.where(oh0, y1, -jnp.inf))
        sx2 = jnp.max(jnp.where(oh0, x2, -jnp.inf))
        sy2 = jnp.max(jnp.where(oh0, y2, -jnp.inf))
        sarea = jnp.max(jnp.where(oh0, areas, -jnp.inf))
        xx1 = jnp.maximum(x1, sx1)
        yy1 = jnp.maximum(y1, sy1)
        xx2 = jnp.minimum(x2, sx2)
        yy2 = jnp.minimum(y2, sy2)
        w = jnp.maximum(xx2 - xx1, 0.0)
        h = jnp.maximum(yy2 - yy1, 0.0)
        inter = w * h
        iou = inter / (sarea + areas - inter)
        active = jnp.where(iou <= _NMS_THRESH, active, 0)
        return (active, first, ox1, oy1, ox2, oy2, osc, olab)

    carry = (active0, jnp.int32(0), zf, zf, zf, zf, zf, zi)
    carry = lax.fori_loop(0, _DETS, body, carry)
    _, _, ox1, oy1, ox2, oy2, osc, olab = carry
    ox1_ref[0] = ox1
    oy1_ref[0] = oy1
    ox2_ref[0] = ox2
    oy2_ref[0] = oy2
    osc_ref[0] = osc
    olab_ref[0] = olab


def _run_nms(x1, y1, x2, y2, sc, iv):
    f32 = jnp.float32
    outs = pl.pallas_call(
        _nms_kernel,
        grid=(2,),
        in_specs=[pl.BlockSpec((1, 16, 128), lambda i: (i, 0, 0))] * 6,
        out_specs=[pl.BlockSpec((1, 1, 128), lambda i: (i, 0, 0))] * 6,
        out_shape=[jax.ShapeDtypeStruct((2, 1, 128), f32)] * 5
        + [jax.ShapeDtypeStruct((2, 1, 128), jnp.int32)],
    )(x1, y1, x2, y2, sc, iv)
    return [o[:, 0] for o in outs]


def kernel(regression_heads, classification_heads, anchors):
    n = regression_heads.shape[0]
    reg = jnp.transpose(regression_heads, (0, 2, 3, 1)).reshape(n, -1, 4)
    cls = jnp.transpose(classification_heads, (0, 2, 3, 1)).reshape(
        n, -1, _NUM_CLASSES)

    cx1, cy1, cx2, cy2, csc, civ = [], [], [], [], [], []
    for i in range(n):
        boxes = _clip_boxes(_decode(reg[i], anchors))
        scores_full = jax.nn.sigmoid(cls[i])
        bx, bs, bv = [], [], []
        for label in range(_NUM_CLASSES):
            score = scores_full[:, label].ravel()
            keep_mask = score > _SCORE_THRESH
            masked_score = jnp.where(keep_mask, score, -jnp.inf)
            order = jnp.argsort(-masked_score)[:_TOPK]
            bx.append(boxes[order])
            bs.append(score[order])
            bv.append(keep_mask[order])
        b = jnp.concatenate(bx, axis=0)
        s = jnp.concatenate(bs, axis=0)
        v = jnp.concatenate(bv, axis=0)
        pad = 2048 - b.shape[0]
        b = jnp.pad(b, ((0, pad), (0, 0)))
        s = jnp.pad(s, ((0, pad),))
        v = jnp.pad(v.astype(jnp.int32), ((0, pad),))
        cx1.append(b[:, 0].reshape(16, 128))
        cy1.append(b[:, 1].reshape(16, 128))
        cx2.append(b[:, 2].reshape(16, 128))
        cy2.append(b[:, 3].reshape(16, 128))
        csc.append(s.reshape(16, 128))
        civ.append(v.reshape(16, 128))

    x1 = jnp.stack(cx1)
    y1 = jnp.stack(cy1)
    x2 = jnp.stack(cx2)
    y2 = jnp.stack(cy2)
    sc = jnp.stack(csc)
    iv = jnp.stack(civ)

    ox1, oy1, ox2, oy2, osc, olab = _run_nms(x1, y1, x2, y2, sc, iv)
    out_boxes = jnp.stack((ox1, oy1, ox2, oy2), axis=-1)[:, :_DETS]
    out_scores = osc[:, :_DETS]
    out_labels = olab[:, :_DETS]
    return out_boxes, out_scores, out_labels
